# baseline (speedup 1.0000x reference)
import functools

import jax
import jax.numpy as jnp
from jax.experimental import pallas as pl
from jax.experimental.pallas import tpu as pltpu

N_LEVELS = 8
TILE = 512


def _rq_tile_kernel(x_ref, embeds_ref, cs_ref, inds_ref, qx_ref, nsmall_ref,
                    loss_ref, *, batch):
    i = pl.program_id(0)

    @pl.when(i == 0)
    def _init():
        nsmall_ref[...] = jnp.sum(
            (cs_ref[...] < 1.0).astype(jnp.float32)).reshape(1, 1)
        loss_ref[...] = jnp.zeros((1, 1), jnp.float32)

    x = x_ref[...]
    d = x.shape[1]
    k = embeds_ref.shape[2]
    residual = x
    loss_acc = jnp.float32(0.0)
    cols = jax.lax.broadcasted_iota(jnp.int32, (x.shape[0], k), 1)
    for l in range(N_LEVELS):
        emb = embeds_ref[l]
        e2 = jnp.sum(emb * emb, axis=0)
        scores = jax.lax.dot_general(
            residual, emb, (((1,), (0,)), ((), ())),
            preferred_element_type=jnp.float32)
        rr = jnp.sum(residual * residual, axis=1)
        neg_dist = -(rr[:, None] - 2.0 * scores + e2[None, :])
        ind = jnp.argmax(neg_dist, axis=1)
        loss_acc += jnp.sum(-jnp.max(neg_dist, axis=1))
        onehot = (cols == ind[:, None]).astype(jnp.float32)
        q = jax.lax.dot_general(
            onehot, emb, (((1,), (1,)), ((), ())),
            precision=jax.lax.Precision.HIGHEST,
            preferred_element_type=jnp.float32)
        residual = residual - q
        inds_ref[:, l] = ind
    qx_ref[...] = x - residual
    loss_ref[...] += (loss_acc / jnp.float32(batch * d)).reshape(1, 1)


@jax.jit
def kernel(x, embeds, cluster_sizes):
    b, d = x.shape
    n_levels, _, k = embeds.shape
    grid = (b // TILE,)
    inds, qx, nsmall, loss = pl.pallas_call(
        functools.partial(_rq_tile_kernel, batch=b),
        grid=grid,
        in_specs=[
            pl.BlockSpec((TILE, d), lambda i: (i, 0)),
            pl.BlockSpec((n_levels, d, k), lambda i: (0, 0, 0)),
            pl.BlockSpec((n_levels, k), lambda i: (0, 0)),
        ],
        out_specs=[
            pl.BlockSpec((TILE, n_levels), lambda i: (i, 0)),
            pl.BlockSpec((TILE, d), lambda i: (i, 0)),
            pl.BlockSpec((1, 1), lambda i: (0, 0)),
            pl.BlockSpec((1, 1), lambda i: (0, 0)),
        ],
        out_shape=[
            jax.ShapeDtypeStruct((b, n_levels), jnp.int32),
            jax.ShapeDtypeStruct((b, d), jnp.float32),
            jax.ShapeDtypeStruct((1, 1), jnp.float32),
            jax.ShapeDtypeStruct((1, 1), jnp.float32),
        ],
        compiler_params=pltpu.CompilerParams(
            dimension_semantics=("arbitrary",)),
    )(x, embeds, cluster_sizes)
    return (inds.astype(jnp.int64), qx, nsmall.reshape(()), loss.reshape(()))


# speedup vs baseline: 1.5931x; 1.5931x over previous
import functools

import jax
import jax.numpy as jnp
from jax.experimental import pallas as pl
from jax.experimental.pallas import tpu as pltpu

N_LEVELS = 8
TILE = 512


def _rq_tile_kernel(x_ref, embeds_ref, cs_ref, inds_ref, qx_ref, nsmall_ref,
                    loss_ref, ehi_ref, emid_ref, elo_ref, e2_ref, *, batch):
    i = pl.program_id(0)

    @pl.when(i == 0)
    def _init():
        nsmall_ref[...] = jnp.sum(
            (cs_ref[...] < 1.0).astype(jnp.float32)).reshape(1, 1)
        loss_ref[...] = jnp.zeros((1, 1), jnp.float32)
        emb_all = embeds_ref[...]
        hi = emb_all.astype(jnp.bfloat16)
        rem = emb_all - hi.astype(jnp.float32)
        mid = rem.astype(jnp.bfloat16)
        lo = (rem - mid.astype(jnp.float32)).astype(jnp.bfloat16)
        ehi_ref[...] = hi
        emid_ref[...] = mid
        elo_ref[...] = lo
        e2_ref[...] = jnp.sum(emb_all * emb_all, axis=1)

    x = x_ref[...]
    d = x.shape[1]
    k = embeds_ref.shape[2]
    residual = x
    loss_acc = jnp.float32(0.0)
    cols = jax.lax.broadcasted_iota(jnp.int32, (x.shape[0], k), 1)
    for l in range(N_LEVELS):
        emb = embeds_ref[l]
        scores = jax.lax.dot_general(
            residual, emb, (((1,), (0,)), ((), ())),
            preferred_element_type=jnp.float32)
        rr = jnp.sum(residual * residual, axis=1)
        neg_dist = -(rr[:, None] - 2.0 * scores + e2_ref[l][None, :])
        ind = jnp.argmax(neg_dist, axis=1)
        loss_acc += jnp.sum(-jnp.max(neg_dist, axis=1))
        onehot = (cols == ind[:, None]).astype(jnp.bfloat16)
        dn = (((1,), (1,)), ((), ()))
        q_hi = jax.lax.dot_general(onehot, ehi_ref[l], dn,
                                   preferred_element_type=jnp.float32)
        q_mid = jax.lax.dot_general(onehot, emid_ref[l], dn,
                                    preferred_element_type=jnp.float32)
        q_lo = jax.lax.dot_general(onehot, elo_ref[l], dn,
                                   preferred_element_type=jnp.float32)
        residual = residual - ((q_hi + q_mid) + q_lo)
        inds_ref[:, l] = ind
    qx_ref[...] = x - residual
    loss_ref[...] += (loss_acc / jnp.float32(batch * d)).reshape(1, 1)


@jax.jit
def kernel(x, embeds, cluster_sizes):
    b, d = x.shape
    n_levels, _, k = embeds.shape
    grid = (b // TILE,)
    inds, qx, nsmall, loss = pl.pallas_call(
        functools.partial(_rq_tile_kernel, batch=b),
        grid=grid,
        in_specs=[
            pl.BlockSpec((TILE, d), lambda i: (i, 0)),
            pl.BlockSpec((n_levels, d, k), lambda i: (0, 0, 0)),
            pl.BlockSpec((n_levels, k), lambda i: (0, 0)),
        ],
        out_specs=[
            pl.BlockSpec((TILE, n_levels), lambda i: (i, 0)),
            pl.BlockSpec((TILE, d), lambda i: (i, 0)),
            pl.BlockSpec((1, 1), lambda i: (0, 0)),
            pl.BlockSpec((1, 1), lambda i: (0, 0)),
        ],
        out_shape=[
            jax.ShapeDtypeStruct((b, n_levels), jnp.int32),
            jax.ShapeDtypeStruct((b, d), jnp.float32),
            jax.ShapeDtypeStruct((1, 1), jnp.float32),
            jax.ShapeDtypeStruct((1, 1), jnp.float32),
        ],
        scratch_shapes=[
            pltpu.VMEM((n_levels, d, k), jnp.bfloat16),
            pltpu.VMEM((n_levels, d, k), jnp.bfloat16),
            pltpu.VMEM((n_levels, d, k), jnp.bfloat16),
            pltpu.VMEM((n_levels, k), jnp.float32),
        ],
        compiler_params=pltpu.CompilerParams(
            dimension_semantics=("arbitrary",)),
    )(x, embeds, cluster_sizes)
    return (inds.astype(jnp.int64), qx, nsmall.reshape(()), loss.reshape(()))
